# baseline (speedup 1.0000x reference)
import functools

import jax
import jax.numpy as jnp
from jax import lax
from jax.experimental import pallas as pl
from jax.experimental.pallas import tpu as pltpu
from jax.experimental.pallas import tpu_sc as plsc

_B = 16384
_D = 128
_NC = 2
_NS = 16
_NW = _NC * _NS
_CH = 128

_SPLIT = 2
_BS = _B // _SPLIT
_BPW = _BS // _NW
_NCH = _BPW // _CH
_NJ = 2 * _NCH

_EO = 300
_EOP = 384
_RB = 4096
_NRB = _BS // _RB
_TCH = 256


def _gather_body(row0_base, utab, itab, uidx, iidx, out_u, out_v, *rest):
    uidx_v, iidx_v = rest[0], rest[1]
    bufs = rest[2:2 + _NJ]
    gsem = rest[2 + _NJ:2 + 2 * _NJ]
    osem = rest[2 + 2 * _NJ:2 + 3 * _NJ]
    wid = lax.axis_index("s") * _NC + lax.axis_index("c")
    row0 = row0_base + wid * _NCH
    iu = pltpu.async_copy(uidx.at[pl.ds(row0, _NCH)], uidx_v, gsem[0])
    ii = pltpu.async_copy(iidx.at[pl.ds(row0, _NCH)], iidx_v, osem[0])
    iu.wait()
    ii.wait()
    base = wid * _BPW
    jobs = ([(utab, uidx_v, out_u, j) for j in range(_NCH)]
            + [(itab, iidx_v, out_v, j) for j in range(_NCH)])
    n = len(jobs)

    def fire(k):
        tab, idxv, _, j = jobs[k]
        return pltpu.async_copy(tab.at[idxv.at[j]], bufs[k], gsem[k])

    gathers = [None] * n
    gathers[0] = fire(0)
    outs = []
    for k in range(n):
        gathers[k].wait()
        if k + 1 < n:
            gathers[k + 1] = fire(k + 1)
        _, _, out, j = jobs[k]
        outs.append(pltpu.async_copy(
            bufs[k], out.at[pl.ds(base + j * _CH, _CH)], osem[k]))
    for o in outs:
        o.wait()


@functools.cache
def _make_gather(row0_base):
    return functools.partial(
        pl.kernel,
        mesh=plsc.VectorSubcoreMesh(core_axis_name="c", subcore_axis_name="s"),
        out_type=[jax.ShapeDtypeStruct((_BS, _D), jnp.float32),
                  jax.ShapeDtypeStruct((_BS, _D), jnp.float32)],
        scratch_types=(
            [pltpu.VMEM((_NCH, _CH), jnp.int32)] * 2
            + [pltpu.VMEM((_CH, _D), jnp.float32)] * _NJ
            + [pltpu.SemaphoreType.DMA] * (2 * _NJ)
        ),
    )(functools.partial(_gather_body, row0_base))


def _dense_body(ue_ref, ve_ref, w_ref, b_ref, eye_ref, *rest):
    out_ref = rest[-1]
    u = jnp.dot(ue_ref[...], w_ref[...],
                preferred_element_type=jnp.float32) + b_ref[...]
    v = jnp.dot(ve_ref[...], w_ref[...],
                preferred_element_type=jnp.float32) + b_ref[...]
    u = jnp.maximum(u, 0.0)
    v = jnp.maximum(v, 0.0)
    num = jnp.sum(u * v, axis=1, keepdims=True)
    den = jnp.sqrt(jnp.sum(u * u, axis=1, keepdims=True)
                   * jnp.sum(v * v, axis=1, keepdims=True))
    s_col = num / jnp.maximum(den, 1e-8)
    hi = s_col.astype(jnp.bfloat16).astype(jnp.float32)
    lo = s_col - hi
    eye = eye_ref[...]
    rows = [
        (jax.lax.dot_general(
            hi[i * _TCH:(i + 1) * _TCH, :], eye,
            (((0,), (0,)), ((), ())),
            preferred_element_type=jnp.float32)
         + jax.lax.dot_general(
            lo[i * _TCH:(i + 1) * _TCH, :], eye,
            (((0,), (0,)), ((), ())),
            preferred_element_type=jnp.float32)).reshape(2, _CH)
        for i in range(_RB // _TCH)
    ]
    out_ref[...] = jnp.concatenate(rows, axis=0)


@functools.cache
def _make_dense(block0, aliased):
    rpb = _RB // _CH
    in_specs = [
        pl.BlockSpec((_RB, _D), lambda i: (i, 0)),
        pl.BlockSpec((_RB, _D), lambda i: (i, 0)),
        pl.BlockSpec((_D, _EOP), lambda i: (0, 0)),
        pl.BlockSpec((1, _EOP), lambda i: (0, 0)),
        pl.BlockSpec((_TCH, _TCH), lambda i: (0, 0)),
    ]
    if aliased:
        in_specs.append(pl.BlockSpec(memory_space=pl.ANY))
    return pl.pallas_call(
        _dense_body,
        grid=(_NRB,),
        in_specs=in_specs,
        out_specs=pl.BlockSpec((rpb, _CH), lambda i: (i + block0, 0)),
        out_shape=jax.ShapeDtypeStruct((_B // _CH, _CH), jnp.float32),
        input_output_aliases={5: 0} if aliased else {},
        compiler_params=pltpu.CompilerParams(
            dimension_semantics=("parallel",)),
    )


def kernel(user_indices, item_indices, user_table, item_table, W, b):
    uidx = user_indices.astype(jnp.int32).reshape(_B // _CH, _CH)
    iidx = item_indices.astype(jnp.int32).reshape(_B // _CH, _CH)
    wp = jnp.pad(W, ((0, 0), (0, _EOP - _EO)))
    bp = jnp.pad(b, (0, _EOP - _EO)).reshape(1, _EOP)
    eye = jnp.eye(_TCH, dtype=jnp.float32)
    embeds = [_make_gather(s * (_BS // _CH))(user_table, item_table,
                                             uidx, iidx)
              for s in range(_SPLIT)]
    scores = None
    for s, (ue, ve) in enumerate(embeds):
        block0 = s * _NRB
        if scores is None:
            scores = _make_dense(block0, False)(ue, ve, wp, bp, eye)
        else:
            scores = _make_dense(block0, True)(ue, ve, wp, bp, eye, scores)
    return scores.reshape(_B)


# speedup vs baseline: 1.0807x; 1.0807x over previous
import functools

import jax
import jax.numpy as jnp
from jax import lax
from jax.experimental import pallas as pl
from jax.experimental.pallas import tpu as pltpu
from jax.experimental.pallas import tpu_sc as plsc

_B = 16384
_D = 128
_NC = 2
_NS = 16
_NW = _NC * _NS
_CH = 128

_SPLIT = 2
_BS = _B // _SPLIT
_BPW = _BS // _NW
_NCH = _BPW // _CH
_NJ = 2 * _NCH

_EO = 300
_EOP = 384
_RB = 4096
_NRB = _BS // _RB
_TCH = 256


def _gather_body(row0_base, utab, itab, uidx, iidx, out_u, out_v, *rest):
    uidx_v, iidx_v = rest[0], rest[1]
    bufs = rest[2:2 + _NJ]
    gsem = rest[2 + _NJ:2 + 2 * _NJ]
    osem = rest[2 + 2 * _NJ:2 + 3 * _NJ]
    wid = lax.axis_index("s") * _NC + lax.axis_index("c")
    row0 = row0_base + wid * _NCH
    iu = pltpu.async_copy(uidx.at[pl.ds(row0, _NCH)], uidx_v, gsem[0])
    ii = pltpu.async_copy(iidx.at[pl.ds(row0, _NCH)], iidx_v, osem[0])
    iu.wait()
    ii.wait()
    base = wid * _BPW
    jobs = ([(utab, uidx_v, out_u, j) for j in range(_NCH)]
            + [(itab, iidx_v, out_v, j) for j in range(_NCH)])
    gathers = [
        pltpu.async_copy(tab.at[idxv.at[j]], bufs[k], gsem[k])
        for k, (tab, idxv, _, j) in enumerate(jobs)
    ]
    outs = []
    for k, (_, _, out, j) in enumerate(jobs):
        gathers[k].wait()
        outs.append(pltpu.async_copy(
            bufs[k], out.at[pl.ds(base + j * _CH, _CH)], osem[k]))
    for o in outs:
        o.wait()


@functools.cache
def _make_gather(row0_base):
    return functools.partial(
        pl.kernel,
        mesh=plsc.VectorSubcoreMesh(core_axis_name="c", subcore_axis_name="s"),
        out_type=[jax.ShapeDtypeStruct((_BS, _D), jnp.float32),
                  jax.ShapeDtypeStruct((_BS, _D), jnp.float32)],
        scratch_types=(
            [pltpu.VMEM((_NCH, _CH), jnp.int32)] * 2
            + [pltpu.VMEM((_CH, _D), jnp.float32)] * _NJ
            + [pltpu.SemaphoreType.DMA] * (2 * _NJ)
        ),
    )(functools.partial(_gather_body, row0_base))


def _dense_body(ue_ref, ve_ref, w_ref, b_ref, eye_ref, *rest):
    out_ref = rest[-1]
    u = jnp.dot(ue_ref[...], w_ref[...],
                preferred_element_type=jnp.float32) + b_ref[...]
    v = jnp.dot(ve_ref[...], w_ref[...],
                preferred_element_type=jnp.float32) + b_ref[...]
    u = jnp.maximum(u, 0.0)
    v = jnp.maximum(v, 0.0)
    num = jnp.sum(u * v, axis=1, keepdims=True)
    den = jnp.sqrt(jnp.sum(u * u, axis=1, keepdims=True)
                   * jnp.sum(v * v, axis=1, keepdims=True))
    s_col = num / jnp.maximum(den, 1e-8)
    hi = s_col.astype(jnp.bfloat16).astype(jnp.float32)
    lo = s_col - hi
    eye = eye_ref[...]
    rows = [
        (jax.lax.dot_general(
            hi[i * _TCH:(i + 1) * _TCH, :], eye,
            (((0,), (0,)), ((), ())),
            preferred_element_type=jnp.float32)
         + jax.lax.dot_general(
            lo[i * _TCH:(i + 1) * _TCH, :], eye,
            (((0,), (0,)), ((), ())),
            preferred_element_type=jnp.float32)).reshape(2, _CH)
        for i in range(_RB // _TCH)
    ]
    out_ref[...] = jnp.concatenate(rows, axis=0)


@functools.cache
def _make_dense(block0, aliased):
    rpb = _RB // _CH
    in_specs = [
        pl.BlockSpec((_RB, _D), lambda i: (i, 0)),
        pl.BlockSpec((_RB, _D), lambda i: (i, 0)),
        pl.BlockSpec((_D, _EOP), lambda i: (0, 0)),
        pl.BlockSpec((1, _EOP), lambda i: (0, 0)),
        pl.BlockSpec((_TCH, _TCH), lambda i: (0, 0)),
    ]
    if aliased:
        in_specs.append(pl.BlockSpec(memory_space=pl.ANY))
    return pl.pallas_call(
        _dense_body,
        grid=(_NRB,),
        in_specs=in_specs,
        out_specs=pl.BlockSpec((rpb, _CH), lambda i: (i + block0, 0)),
        out_shape=jax.ShapeDtypeStruct((_B // _CH, _CH), jnp.float32),
        input_output_aliases={5: 0} if aliased else {},
        compiler_params=pltpu.CompilerParams(
            dimension_semantics=("parallel",)),
    )


def kernel(user_indices, item_indices, user_table, item_table, W, b):
    uidx = user_indices.astype(jnp.int32).reshape(_B // _CH, _CH)
    iidx = item_indices.astype(jnp.int32).reshape(_B // _CH, _CH)
    wp = jnp.pad(W, ((0, 0), (0, _EOP - _EO)))
    bp = jnp.pad(b, (0, _EOP - _EO)).reshape(1, _EOP)
    eye = jnp.eye(_TCH, dtype=jnp.float32)
    embeds = [_make_gather(s * (_BS // _CH))(user_table, item_table,
                                             uidx, iidx)
              for s in range(_SPLIT)]
    scores = None
    for s, (ue, ve) in enumerate(embeds):
        block0 = s * _NRB
        if scores is None:
            scores = _make_dense(block0, False)(ue, ve, wp, bp, eye)
        else:
            scores = _make_dense(block0, True)(ue, ve, wp, bp, eye, scores)
    return scores.reshape(_B)
